# baseline (speedup 1.0000x reference)
import functools

import jax
import jax.numpy as jnp
from jax import lax
from jax.experimental import pallas as pl
from jax.experimental.pallas import tpu as pltpu
from jax.experimental.pallas import tpu_sc as plsc

_SEQ_LEN = 16
_NUM_SEQS = 2
_NUM_LEVELS = 128
_EPS = 1.2e-07
_BLK = 64


def _k1_body(x_ref, sc_ref, wp_ref, wv_ref, g0_ref, g1_ref, r_ref, ins_ref,
             idx_ref, w_ref, *, n_classes):
    blk = x_ref.shape[0]
    xs = x_ref[...] * sc_ref[0, 0]
    logits = jnp.dot(xs, wp_ref[...], preferred_element_type=jnp.float32)
    m = jnp.max(logits, axis=1, keepdims=True)
    e = jnp.exp(logits - m)
    s = jnp.sum(e, axis=1, keepdims=True)
    a = e * (1.0 - n_classes * _EPS) + s * _EPS
    lv = jnp.dot(xs, wv_ref[...], preferred_element_type=jnp.float32)
    iota = jax.lax.broadcasted_iota(jnp.int32, (blk, n_classes), 1)
    idx_cols = []
    lv_cols = []
    for g_ref in (g0_ref, g1_ref):
        keys = a * g_ref[...]
        bound = jnp.full((blk, 1), jnp.inf, jnp.float32)
        for _ in range(_SEQ_LEN):
            mx = jnp.max(jnp.where(keys < bound, keys, -1.0), axis=1,
                         keepdims=True)
            hit = keys == mx
            idx_cols.append(jnp.min(jnp.where(hit, iota, n_classes), axis=1))
            lv_cols.append(jnp.sum(jnp.where(hit, lv, 0.0), axis=1))
            bound = mx
    idx_mat = jnp.stack(idx_cols, axis=1)
    lv_mat = jnp.stack(lv_cols, axis=1)
    r = r_ref[...]
    inv_ns = ins_ref[0, 0]
    w_parts = []
    for s in range(_NUM_SEQS):
        lv16 = lv_mat[:, s * _SEQ_LEN:(s + 1) * _SEQ_LEN]
        mx = jnp.max(lv16, axis=1, keepdims=True)
        ev = jnp.exp(lv16 - mx)
        cv = ev / jnp.sum(ev, axis=1, keepdims=True)
        t = cv * (_NUM_LEVELS - 1.0) + 0.999 * r[:, s * _SEQ_LEN:(s + 1) * _SEQ_LEN]
        disc = jnp.floor(t).astype(jnp.int32).astype(jnp.float32) * (
            1.0 / (_NUM_LEVELS - 1))
        w_parts.append(disc * inv_ns)
    idx_ref[...] = idx_mat
    w_ref[...] = jnp.concatenate(w_parts, axis=1)


_NC = 2
_NS = 16
_NW = _NC * _NS


def _sc_proj_body(wout_hbm, idx_hbm, wflat_hbm, b_hbm, y_hbm,
                  idx_l, wflat_l, b_l, rows_l, yrow_l,
                  sem_g0, sem_g1, sem_s0, sem_s1, *, rows_per, d):
    nk = _NUM_SEQS * _SEQ_LEN
    wid = lax.axis_index("s") * _NC + lax.axis_index("c")
    base = wid * rows_per
    pltpu.sync_copy(idx_hbm.at[pl.ds(base, rows_per)], idx_l)
    pltpu.sync_copy(wflat_hbm.at[pl.ds(base * nk, rows_per * nk)], wflat_l)
    pltpu.sync_copy(b_hbm, b_l)
    nch = d // 16
    sems_g = (sem_g0, sem_g1)
    sems_s = (sem_s0, sem_s1)

    pltpu.async_copy(wout_hbm.at[idx_l.at[0]], rows_l.at[0], sem_g0)
    pltpu.async_copy(wout_hbm.at[idx_l.at[1]], rows_l.at[1], sem_g1)

    def body(i, carry):
        for half in range(2):
            r = 2 * i + half
            sg = sems_g[half]
            ss = sems_s[half]
            pltpu.make_async_copy(wout_hbm.at[idx_l.at[r]],
                                  rows_l.at[half], sg).wait()
            acc = [b_l[pl.ds(c * 16, 16)] for c in range(nch)]
            wrow = [wflat_l[pl.ds(r * nk + 16 * h, 16)] for h in range(nk // 16)]
            for j in range(nk):
                wb = wrow[j // 16][j % 16]
                for c in range(nch):
                    acc[c] = acc[c] + wb * rows_l[half, j, pl.ds(c * 16, 16)]
            @pl.when(r >= 2)
            def _drain():
                pltpu.make_async_copy(yrow_l.at[half],
                                      y_hbm.at[base + r - 2], ss).wait()
            for c in range(nch):
                yrow_l[half, pl.ds(c * 16, 16)] = acc[c]
            pltpu.async_copy(yrow_l.at[half], y_hbm.at[base + r], ss)

            @pl.when(r + 2 < rows_per)
            def _next():
                pltpu.async_copy(wout_hbm.at[idx_l.at[r + 2]],
                                 rows_l.at[half], sg)
        return carry

    lax.fori_loop(0, rows_per // 2, body, 0)
    pltpu.make_async_copy(yrow_l.at[0],
                          y_hbm.at[base + rows_per - 2], sem_s0).wait()
    pltpu.make_async_copy(yrow_l.at[1],
                          y_hbm.at[base + rows_per - 1], sem_s1).wait()


def kernel(x, input_scale, W_probs, W_values, W_out, b_out, num_seqs):
    B, D = x.shape
    N = W_probs.shape[0]
    nblk = B // _BLK

    u = jax.random.uniform(jax.random.key(42), (B, _NUM_SEQS, N),
                           minval=1e-20, maxval=1.0)
    g = -1.0 / jnp.log(u)
    g0 = g[:, 0, :]
    g1 = g[:, 1, :]
    r = jax.random.uniform(jax.random.key(7), (B, _NUM_SEQS, _SEQ_LEN),
                           dtype=jnp.float32).reshape(B, _NUM_SEQS * _SEQ_LEN)

    sc2 = jnp.reshape(input_scale, (1, 1)).astype(jnp.float32)
    inv_ns = jnp.reshape(1.0 / jnp.asarray(num_seqs, jnp.float32), (1, 1))
    wpT = W_probs.T
    wvT = W_values.T
    woutT = W_out.T
    b2 = jnp.reshape(b_out, (1, D))

    nk = _NUM_SEQS * _SEQ_LEN
    idx_mat, w_mat = pl.pallas_call(
        functools.partial(_k1_body, n_classes=N),
        grid=(nblk,),
        in_specs=[
            pl.BlockSpec((_BLK, D), lambda i: (i, 0)),
            pl.BlockSpec((1, 1), lambda i: (0, 0)),
            pl.BlockSpec((D, N), lambda i: (0, 0)),
            pl.BlockSpec((D, N), lambda i: (0, 0)),
            pl.BlockSpec((_BLK, N), lambda i: (i, 0)),
            pl.BlockSpec((_BLK, N), lambda i: (i, 0)),
            pl.BlockSpec((_BLK, nk), lambda i: (i, 0)),
            pl.BlockSpec((1, 1), lambda i: (0, 0)),
        ],
        out_specs=[
            pl.BlockSpec((_BLK, nk), lambda i: (i, 0)),
            pl.BlockSpec((_BLK, nk), lambda i: (i, 0)),
        ],
        out_shape=[
            jax.ShapeDtypeStruct((B, nk), jnp.int32),
            jax.ShapeDtypeStruct((B, nk), jnp.float32),
        ],
    )(x, sc2, wpT, wvT, g0, g1, r, inv_ns)

    rows_per = B // _NW
    sc_proj = pl.kernel(
        functools.partial(_sc_proj_body, rows_per=rows_per, d=D),
        mesh=plsc.VectorSubcoreMesh(core_axis_name="c", subcore_axis_name="s"),
        out_type=jax.ShapeDtypeStruct((B, D), jnp.float32),
        scratch_types=[
            pltpu.VMEM((rows_per, nk), jnp.int32),
            pltpu.VMEM((rows_per * nk,), jnp.float32),
            pltpu.VMEM((D,), jnp.float32),
            pltpu.VMEM((2, nk, D), jnp.float32),
            pltpu.VMEM((2, D), jnp.float32),
            pltpu.SemaphoreType.DMA,
            pltpu.SemaphoreType.DMA,
            pltpu.SemaphoreType.DMA,
            pltpu.SemaphoreType.DMA,
        ],
    )
    y = sc_proj(woutT, idx_mat, w_mat.reshape(B * nk), b_out)
    return y


# speedup vs baseline: 1.0275x; 1.0275x over previous
import functools

import jax
import jax.numpy as jnp
from jax import lax
from jax.experimental import pallas as pl
from jax.experimental.pallas import tpu as pltpu
from jax.experimental.pallas import tpu_sc as plsc

_SEQ_LEN = 16
_NUM_SEQS = 2
_NUM_LEVELS = 128
_EPS = 1.2e-07
_BLK = 32


def _band16(vals, idxs, lvs, n_classes):
    blk = vals.shape[0]
    idx_cols = []
    lv_cols = []
    bound = jnp.full((blk, 1), jnp.inf, jnp.float32)
    for _ in range(_SEQ_LEN):
        mx = jnp.max(jnp.where(vals < bound, vals, -1.0), axis=1,
                     keepdims=True)
        hit = vals == mx
        idx_cols.append(jnp.min(jnp.where(hit, idxs, n_classes), axis=1))
        lv_cols.append(jnp.sum(jnp.where(hit, lvs, 0.0), axis=1))
        bound = mx
    return idx_cols, lv_cols, bound


def _cand16(keys, lv, n_classes):
    blk = keys.shape[0]
    gd = n_classes // (16 * 128)
    k4 = keys.reshape(blk, gd, 16, 128)
    lv4 = lv.reshape(blk, gd, 16, 128)
    s_iota = jax.lax.broadcasted_iota(jnp.int32, (blk, gd, 16, 128), 2)
    g_iota = jax.lax.broadcasted_iota(jnp.int32, (blk, gd, 128), 1)
    l_iota = jax.lax.broadcasted_iota(jnp.int32, (blk, gd, 128), 2)
    ms, iss, vs = [], [], []
    cur = k4
    for _tier in range(3):
        mt = jnp.max(cur, axis=2)
        hit = cur == mt[:, :, None, :]
        st = jnp.min(jnp.where(hit, s_iota, 16), axis=2)
        vt = jnp.sum(jnp.where(hit, lv4, 0.0), axis=2)
        ms.append(mt)
        iss.append((g_iota * 16 + st) * 128 + l_iota)
        vs.append(vt)
        cur = jnp.where(hit, -1.0, cur)
    m4 = jnp.max(cur, axis=2)
    nc = 3 * gd * 128
    mc = jnp.concatenate(ms, axis=1).reshape(blk, nc)
    ic = jnp.concatenate(iss, axis=1).reshape(blk, nc)
    vc = jnp.concatenate(vs, axis=1).reshape(blk, nc)
    idx_cols, lv_cols, b16 = _band16(mc, ic, vc, n_classes)
    bad = jnp.max(jnp.where(m4 >= b16[:, :, None], 1, 0))
    return idx_cols, lv_cols, bad


def _epilogue(idx_cols, lv_cols, r, inv_ns, idx_ref, w_ref):
    idx_mat = jnp.stack(idx_cols, axis=1)
    lv_mat = jnp.stack(lv_cols, axis=1)
    w_parts = []
    for s in range(_NUM_SEQS):
        lv16 = lv_mat[:, s * _SEQ_LEN:(s + 1) * _SEQ_LEN]
        mx = jnp.max(lv16, axis=1, keepdims=True)
        ev = jnp.exp(lv16 - mx)
        cv = ev / jnp.sum(ev, axis=1, keepdims=True)
        t = cv * (_NUM_LEVELS - 1.0) + 0.999 * r[:, s * _SEQ_LEN:(s + 1) * _SEQ_LEN]
        disc = jnp.floor(t).astype(jnp.int32).astype(jnp.float32) * (
            1.0 / (_NUM_LEVELS - 1))
        w_parts.append(disc * inv_ns)
    idx_ref[...] = idx_mat
    w_ref[...] = jnp.concatenate(w_parts, axis=1)


def _k1_body(x_ref, sc_ref, wp_ref, wv_ref, g0_ref, g1_ref, r_ref, ins_ref,
             idx_ref, w_ref, *, n_classes):
    blk = x_ref.shape[0]
    xs = x_ref[...] * sc_ref[0, 0]
    logits = jnp.dot(xs, wp_ref[...], preferred_element_type=jnp.float32)
    m = jnp.max(logits, axis=1, keepdims=True)
    e = jnp.exp(logits - m)
    s = jnp.sum(e, axis=1, keepdims=True)
    a = e * (1.0 - n_classes * _EPS) + s * _EPS
    lv = jnp.dot(xs, wv_ref[...], preferred_element_type=jnp.float32)
    r = r_ref[...]
    inv_ns = ins_ref[0, 0]
    idx_cols = []
    lv_cols = []
    bad = jnp.int32(0)
    for g_ref in (g0_ref, g1_ref):
        ic, lc, b = _cand16(a * g_ref[...], lv, n_classes)
        idx_cols += ic
        lv_cols += lc
        bad = jnp.maximum(bad, b)
    _epilogue(idx_cols, lv_cols, r, inv_ns, idx_ref, w_ref)

    @pl.when(bad > 0)
    def _fallback():
        iota = jax.lax.broadcasted_iota(jnp.int32, (blk, n_classes), 1)
        f_idx, f_lv = [], []
        for g_ref in (g0_ref, g1_ref):
            ic, lc, _ = _band16(a * g_ref[...], iota, lv, n_classes)
            f_idx += ic
            f_lv += lc
        _epilogue(f_idx, f_lv, r, inv_ns, idx_ref, w_ref)


_NC = 2
_NS = 16
_NW = _NC * _NS


def _sc_proj_body(wout_hbm, idx_hbm, wflat_hbm, b_hbm, y_hbm,
                  idx_l, wflat_l, b_l, rows_l, yrow_l,
                  sem_g0, sem_g1, sem_s0, sem_s1, *, rows_per, d):
    nk = _NUM_SEQS * _SEQ_LEN
    wid = lax.axis_index("s") * _NC + lax.axis_index("c")
    base = wid * rows_per
    pltpu.sync_copy(idx_hbm.at[pl.ds(base, rows_per)], idx_l)
    pltpu.sync_copy(wflat_hbm.at[pl.ds(base * nk, rows_per * nk)], wflat_l)
    pltpu.sync_copy(b_hbm, b_l)
    nch = d // 16
    sems_g = (sem_g0, sem_g1)
    sems_s = (sem_s0, sem_s1)

    pltpu.async_copy(wout_hbm.at[idx_l.at[0]], rows_l.at[0], sem_g0)
    pltpu.async_copy(wout_hbm.at[idx_l.at[1]], rows_l.at[1], sem_g1)

    def body(i, carry):
        for half in range(2):
            r = 2 * i + half
            sg = sems_g[half]
            ss = sems_s[half]
            pltpu.make_async_copy(wout_hbm.at[idx_l.at[r]],
                                  rows_l.at[half], sg).wait()
            acc = [b_l[pl.ds(c * 16, 16)] for c in range(nch)]
            wrow = [wflat_l[pl.ds(r * nk + 16 * h, 16)] for h in range(nk // 16)]
            for j in range(nk):
                wb = wrow[j // 16][j % 16]
                for c in range(nch):
                    acc[c] = acc[c] + wb * rows_l[half, j, pl.ds(c * 16, 16)]
            @pl.when(r >= 2)
            def _drain():
                pltpu.make_async_copy(yrow_l.at[half],
                                      y_hbm.at[base + r - 2], ss).wait()
            for c in range(nch):
                yrow_l[half, pl.ds(c * 16, 16)] = acc[c]
            pltpu.async_copy(yrow_l.at[half], y_hbm.at[base + r], ss)

            @pl.when(r + 2 < rows_per)
            def _next():
                pltpu.async_copy(wout_hbm.at[idx_l.at[r + 2]],
                                 rows_l.at[half], sg)
        return carry

    lax.fori_loop(0, rows_per // 2, body, 0)
    pltpu.make_async_copy(yrow_l.at[0],
                          y_hbm.at[base + rows_per - 2], sem_s0).wait()
    pltpu.make_async_copy(yrow_l.at[1],
                          y_hbm.at[base + rows_per - 1], sem_s1).wait()


def kernel(x, input_scale, W_probs, W_values, W_out, b_out, num_seqs):
    B, D = x.shape
    N = W_probs.shape[0]
    nblk = B // _BLK

    u = jax.random.uniform(jax.random.key(42), (B, _NUM_SEQS, N),
                           minval=1e-20, maxval=1.0)
    g = -1.0 / jnp.log(u)
    g0 = g[:, 0, :]
    g1 = g[:, 1, :]
    r = jax.random.uniform(jax.random.key(7), (B, _NUM_SEQS, _SEQ_LEN),
                           dtype=jnp.float32).reshape(B, _NUM_SEQS * _SEQ_LEN)

    sc2 = jnp.reshape(input_scale, (1, 1)).astype(jnp.float32)
    inv_ns = jnp.reshape(1.0 / jnp.asarray(num_seqs, jnp.float32), (1, 1))
    wpT = W_probs.T
    wvT = W_values.T
    woutT = W_out.T
    b2 = jnp.reshape(b_out, (1, D))

    nk = _NUM_SEQS * _SEQ_LEN
    idx_mat, w_mat = pl.pallas_call(
        functools.partial(_k1_body, n_classes=N),
        grid=(nblk,),
        in_specs=[
            pl.BlockSpec((_BLK, D), lambda i: (i, 0)),
            pl.BlockSpec((1, 1), lambda i: (0, 0)),
            pl.BlockSpec((D, N), lambda i: (0, 0)),
            pl.BlockSpec((D, N), lambda i: (0, 0)),
            pl.BlockSpec((_BLK, N), lambda i: (i, 0)),
            pl.BlockSpec((_BLK, N), lambda i: (i, 0)),
            pl.BlockSpec((_BLK, nk), lambda i: (i, 0)),
            pl.BlockSpec((1, 1), lambda i: (0, 0)),
        ],
        out_specs=[
            pl.BlockSpec((_BLK, nk), lambda i: (i, 0)),
            pl.BlockSpec((_BLK, nk), lambda i: (i, 0)),
        ],
        out_shape=[
            jax.ShapeDtypeStruct((B, nk), jnp.int32),
            jax.ShapeDtypeStruct((B, nk), jnp.float32),
        ],
    )(x, sc2, wpT, wvT, g0, g1, r, inv_ns)

    rows_per = B // _NW
    sc_proj = pl.kernel(
        functools.partial(_sc_proj_body, rows_per=rows_per, d=D),
        mesh=plsc.VectorSubcoreMesh(core_axis_name="c", subcore_axis_name="s"),
        out_type=jax.ShapeDtypeStruct((B, D), jnp.float32),
        scratch_types=[
            pltpu.VMEM((rows_per, nk), jnp.int32),
            pltpu.VMEM((rows_per * nk,), jnp.float32),
            pltpu.VMEM((D,), jnp.float32),
            pltpu.VMEM((2, nk, D), jnp.float32),
            pltpu.VMEM((2, D), jnp.float32),
            pltpu.SemaphoreType.DMA,
            pltpu.SemaphoreType.DMA,
            pltpu.SemaphoreType.DMA,
            pltpu.SemaphoreType.DMA,
        ],
    )
    y = sc_proj(woutT, idx_mat, w_mat.reshape(B * nk), b_out)
    return y


# speedup vs baseline: 1.0482x; 1.0201x over previous
import functools

import jax
import jax.numpy as jnp
from jax import lax
from jax.experimental import pallas as pl
from jax.experimental.pallas import tpu as pltpu
from jax.experimental.pallas import tpu_sc as plsc

_SEQ_LEN = 16
_NUM_SEQS = 2
_NUM_LEVELS = 128
_EPS = 1.2e-07
_BLK = 32


def _band16(vals, idxs, lvs, n_classes):
    blk = vals.shape[0]
    idx_cols = []
    lv_cols = []
    bound = jnp.full((blk, 1), jnp.inf, jnp.float32)
    for _ in range(_SEQ_LEN):
        mx = jnp.max(jnp.where(vals < bound, vals, -1.0), axis=1,
                     keepdims=True)
        hit = vals == mx
        idx_cols.append(jnp.min(jnp.where(hit, idxs, n_classes), axis=1))
        lv_cols.append(jnp.sum(jnp.where(hit, lvs, 0.0), axis=1))
        bound = mx
    return idx_cols, lv_cols, bound


def _cand16(keys, lv, n_classes):
    blk = keys.shape[0]
    gd = n_classes // (16 * 128)
    k4 = keys.reshape(blk, gd, 16, 128)
    lv4 = lv.reshape(blk, gd, 16, 128)
    s_iota = jax.lax.broadcasted_iota(jnp.int32, (blk, gd, 16, 128), 2)
    g_iota = jax.lax.broadcasted_iota(jnp.int32, (blk, gd, 128), 1)
    l_iota = jax.lax.broadcasted_iota(jnp.int32, (blk, gd, 128), 2)
    ms, iss, vs = [], [], []
    cur = k4
    for _tier in range(3):
        mt = jnp.max(cur, axis=2)
        hit = cur == mt[:, :, None, :]
        st = jnp.min(jnp.where(hit, s_iota, 16), axis=2)
        vt = jnp.sum(jnp.where(hit, lv4, 0.0), axis=2)
        ms.append(mt)
        iss.append((g_iota * 16 + st) * 128 + l_iota)
        vs.append(vt)
        cur = jnp.where(hit, -1.0, cur)
    m4 = jnp.max(cur, axis=2)
    nc = 3 * gd * 128
    mc = jnp.concatenate(ms, axis=1).reshape(blk, nc)
    ic = jnp.concatenate(iss, axis=1).reshape(blk, nc)
    vc = jnp.concatenate(vs, axis=1).reshape(blk, nc)
    idx_cols, lv_cols, b16 = _band16(mc, ic, vc, n_classes)
    bad = jnp.max(jnp.where(m4 >= b16[:, :, None], 1, 0))
    return idx_cols, lv_cols, bad


def _epilogue(idx_cols, lv_cols, r, inv_ns, idx_ref, w_ref):
    idx_mat = jnp.stack(idx_cols, axis=1)
    lv_mat = jnp.stack(lv_cols, axis=1)
    w_parts = []
    for s in range(_NUM_SEQS):
        lv16 = lv_mat[:, s * _SEQ_LEN:(s + 1) * _SEQ_LEN]
        mx = jnp.max(lv16, axis=1, keepdims=True)
        ev = jnp.exp(lv16 - mx)
        cv = ev / jnp.sum(ev, axis=1, keepdims=True)
        t = cv * (_NUM_LEVELS - 1.0) + 0.999 * r[:, s * _SEQ_LEN:(s + 1) * _SEQ_LEN]
        disc = jnp.floor(t).astype(jnp.int32).astype(jnp.float32) * (
            1.0 / (_NUM_LEVELS - 1))
        w_parts.append(disc * inv_ns)
    idx_ref[...] = idx_mat
    w_ref[...] = jnp.concatenate(w_parts, axis=1)


def _k1_body(x_ref, sc_ref, wp_ref, wv_ref, g0_ref, g1_ref, r_ref, ins_ref,
             idx_ref, w_ref, *, n_classes):
    blk = x_ref.shape[0]
    xs = x_ref[...] * sc_ref[0, 0]
    logits = jnp.dot(xs, wp_ref[...], preferred_element_type=jnp.float32)
    m = jnp.max(logits, axis=1, keepdims=True)
    e = jnp.exp(logits - m)
    s = jnp.sum(e, axis=1, keepdims=True)
    a = e * (1.0 - n_classes * _EPS) + s * _EPS
    lv = jnp.dot(xs, wv_ref[...], preferred_element_type=jnp.float32)
    r = r_ref[...]
    inv_ns = ins_ref[0, 0]
    idx_cols = []
    lv_cols = []
    bad = jnp.int32(0)
    for g_ref in (g0_ref, g1_ref):
        ic, lc, b = _cand16(a * g_ref[...], lv, n_classes)
        idx_cols += ic
        lv_cols += lc
        bad = jnp.maximum(bad, b)
    _epilogue(idx_cols, lv_cols, r, inv_ns, idx_ref, w_ref)

    @pl.when(bad > 0)
    def _fallback():
        iota = jax.lax.broadcasted_iota(jnp.int32, (blk, n_classes), 1)
        f_idx, f_lv = [], []
        for g_ref in (g0_ref, g1_ref):
            ic, lc, _ = _band16(a * g_ref[...], iota, lv, n_classes)
            f_idx += ic
            f_lv += lc
        _epilogue(f_idx, f_lv, r, inv_ns, idx_ref, w_ref)


_NC = 2
_NS = 16
_NW = _NC * _NS


def _sc_proj_body(wout_hbm, idx_hbm, wflat_hbm, b_hbm, y_hbm,
                  idx_l, wflat_l, b_l, rows_l, yrow_l,
                  sem_g0, sem_g1, sem_s0, sem_s1, *, rows_per, d):
    nk = _NUM_SEQS * _SEQ_LEN
    wid = lax.axis_index("s") * _NC + lax.axis_index("c")
    base = wid * rows_per
    pltpu.sync_copy(idx_hbm.at[pl.ds(base, rows_per)], idx_l)
    pltpu.sync_copy(wflat_hbm.at[pl.ds(base * nk, rows_per * nk)], wflat_l)
    pltpu.sync_copy(b_hbm, b_l)
    nch = d // 16
    sems_g = (sem_g0, sem_g1)
    sems_s = (sem_s0, sem_s1)

    pltpu.async_copy(wout_hbm.at[idx_l.at[0]], rows_l.at[0], sem_g0)
    pltpu.async_copy(wout_hbm.at[idx_l.at[1]], rows_l.at[1], sem_g1)

    def body(i, carry):
        for half in range(2):
            r = 2 * i + half
            sg = sems_g[half]
            ss = sems_s[half]
            pltpu.make_async_copy(wout_hbm.at[idx_l.at[r]],
                                  rows_l.at[half], sg).wait()
            acc = [b_l[pl.ds(c * 16, 16)] for c in range(nch)]
            wrow = [wflat_l[pl.ds(r * nk + 16 * h, 16)] for h in range(nk // 16)]
            for j in range(nk):
                wb = wrow[j // 16][j % 16]
                for c in range(nch):
                    acc[c] = acc[c] + wb * rows_l[half, j, pl.ds(c * 16, 16)]
            @pl.when(r >= 2)
            def _drain():
                pltpu.make_async_copy(yrow_l.at[half],
                                      y_hbm.at[base + r - 2], ss).wait()
            for c in range(nch):
                yrow_l[half, pl.ds(c * 16, 16)] = acc[c]
            pltpu.async_copy(yrow_l.at[half], y_hbm.at[base + r], ss)

            @pl.when(r + 2 < rows_per)
            def _next():
                pltpu.async_copy(wout_hbm.at[idx_l.at[r + 2]],
                                 rows_l.at[half], sg)
        return carry

    lax.fori_loop(0, rows_per // 2, body, 0)
    pltpu.make_async_copy(yrow_l.at[0],
                          y_hbm.at[base + rows_per - 2], sem_s0).wait()
    pltpu.make_async_copy(yrow_l.at[1],
                          y_hbm.at[base + rows_per - 1], sem_s1).wait()


def kernel(x, input_scale, W_probs, W_values, W_out, b_out, num_seqs):
    B, D = x.shape
    N = W_probs.shape[0]
    nblk = B // _BLK

    u = jax.random.uniform(jax.random.key(42), (B, _NUM_SEQS, N),
                           minval=1e-20, maxval=1.0)
    g = -1.0 / jnp.log(u)
    g0 = g[:, 0, :]
    g1 = g[:, 1, :]
    r = jax.random.uniform(jax.random.key(7), (B, _NUM_SEQS, _SEQ_LEN),
                           dtype=jnp.float32).reshape(B, _NUM_SEQS * _SEQ_LEN)

    sc2 = jnp.reshape(input_scale, (1, 1)).astype(jnp.float32)
    inv_ns = jnp.reshape(1.0 / jnp.asarray(num_seqs, jnp.float32), (1, 1))
    wpT = W_probs.T
    wvT = W_values.T
    woutT = W_out.T
    b2 = jnp.reshape(b_out, (1, D))

    nk = _NUM_SEQS * _SEQ_LEN
    nparts = 4
    bp = B // nparts
    rows_per = bp // _NW
    sc_proj = pl.kernel(
        functools.partial(_sc_proj_body, rows_per=rows_per, d=D),
        mesh=plsc.VectorSubcoreMesh(core_axis_name="c", subcore_axis_name="s"),
        out_type=jax.ShapeDtypeStruct((bp, D), jnp.float32),
        scratch_types=[
            pltpu.VMEM((rows_per, nk), jnp.int32),
            pltpu.VMEM((rows_per * nk,), jnp.float32),
            pltpu.VMEM((D,), jnp.float32),
            pltpu.VMEM((2, nk, D), jnp.float32),
            pltpu.VMEM((2, D), jnp.float32),
            pltpu.SemaphoreType.DMA,
            pltpu.SemaphoreType.DMA,
            pltpu.SemaphoreType.DMA,
            pltpu.SemaphoreType.DMA,
        ],
    )
    k1 = pl.pallas_call(
        functools.partial(_k1_body, n_classes=N),
        grid=(bp // _BLK,),
        in_specs=[
            pl.BlockSpec((_BLK, D), lambda i: (i, 0)),
            pl.BlockSpec((1, 1), lambda i: (0, 0)),
            pl.BlockSpec((D, N), lambda i: (0, 0)),
            pl.BlockSpec((D, N), lambda i: (0, 0)),
            pl.BlockSpec((_BLK, N), lambda i: (i, 0)),
            pl.BlockSpec((_BLK, N), lambda i: (i, 0)),
            pl.BlockSpec((_BLK, nk), lambda i: (i, 0)),
            pl.BlockSpec((1, 1), lambda i: (0, 0)),
        ],
        out_specs=[
            pl.BlockSpec((_BLK, nk), lambda i: (i, 0)),
            pl.BlockSpec((_BLK, nk), lambda i: (i, 0)),
        ],
        out_shape=[
            jax.ShapeDtypeStruct((bp, nk), jnp.int32),
            jax.ShapeDtypeStruct((bp, nk), jnp.float32),
        ],
    )
    y_parts = []
    for p in range(nparts):
        sl = slice(p * bp, (p + 1) * bp)
        idx_mat, w_mat = k1(x[sl], sc2, wpT, wvT, g0[sl], g1[sl], r[sl],
                            inv_ns)
        y_parts.append(sc_proj(woutT, idx_mat, w_mat.reshape(bp * nk), b_out))
    return jnp.concatenate(y_parts, axis=0)
